# baseline (speedup 1.0000x reference)
import jax
import jax.numpy as jnp
import numpy as np
from jax.experimental import pallas as pl
from jax.experimental.pallas import tpu as pltpu

_C, _T, _H, _W = 3, 64, 224, 224
_TS = 16
_IDX = [int(x) for x in np.linspace(0.0, float(_T - 1), _TS).astype(np.int64)]
_NCHUNK = 8
_FPC = _T // _NCHUNK


def _body(f, slow, fast, sem_fast, sem_slow):
    copies = []
    for k in range(_NCHUNK):
        c = pltpu.make_async_copy(
            f.at[:, pl.ds(k * _FPC, _FPC)],
            fast.at[:, pl.ds(k * _FPC, _FPC)],
            sem_fast,
        )
        c.start()
        copies.append(c)
    for i, t in enumerate(_IDX):
        c = pltpu.make_async_copy(f.at[:, t], slow.at[:, i], sem_slow)
        c.start()
        copies.append(c)
    for c in copies:
        c.wait()


_pack = pl.pallas_call(
    _body,
    out_shape=(
        jax.ShapeDtypeStruct((_C, _TS, _H, _W), jnp.float32),
        jax.ShapeDtypeStruct((_C, _T, _H, _W), jnp.float32),
    ),
    in_specs=[pl.BlockSpec(memory_space=pl.ANY)],
    out_specs=(
        pl.BlockSpec(memory_space=pl.ANY),
        pl.BlockSpec(memory_space=pl.ANY),
    ),
    scratch_shapes=[pltpu.SemaphoreType.DMA, pltpu.SemaphoreType.DMA],
)


def kernel(frames):
    return _pack(frames)


# speedup vs baseline: 48.3816x; 48.3816x over previous
import jax
import jax.numpy as jnp
from jax.experimental import pallas as pl

_C, _T, _H, _W = 3, 64, 224, 224
_TS = 16
_FPC = _T // _TS


def _body(x_ref, slow_ref, fast_ref):
    fast_ref[...] = x_ref[...]
    i = pl.program_id(0)
    off = (63 * i) // 15 - _FPC * i
    slow_ref[...] = x_ref[:, pl.ds(off, 1)]


_pack = pl.pallas_call(
    _body,
    grid=(_TS,),
    in_specs=[
        pl.BlockSpec((_C, _FPC, _H, _W), lambda i: (0, i, 0, 0)),
    ],
    out_specs=(
        pl.BlockSpec((_C, 1, _H, _W), lambda i: (0, i, 0, 0)),
        pl.BlockSpec((_C, _FPC, _H, _W), lambda i: (0, i, 0, 0)),
    ),
    out_shape=(
        jax.ShapeDtypeStruct((_C, _TS, _H, _W), jnp.float32),
        jax.ShapeDtypeStruct((_C, _T, _H, _W), jnp.float32),
    ),
)


def kernel(frames):
    return _pack(frames)


# speedup vs baseline: 51.2997x; 1.0603x over previous
import jax
import jax.numpy as jnp
from jax.experimental import pallas as pl

_C, _T, _H, _W = 3, 64, 224, 224
_TS = 16
_NG = 8
_FPC = _T // _NG
_SPC = _TS // _NG


def _body(x_ref, slow_ref, fast_ref):
    fast_ref[...] = x_ref[...]
    k = pl.program_id(0)
    for j in range(_SPC):
        i = _SPC * k + j
        off = (63 * i) // 15 - _FPC * k
        slow_ref[:, pl.ds(j, 1)] = x_ref[:, pl.ds(off, 1)]


_pack = pl.pallas_call(
    _body,
    grid=(_NG,),
    in_specs=[
        pl.BlockSpec((_C, _FPC, _H, _W), lambda i: (0, i, 0, 0)),
    ],
    out_specs=(
        pl.BlockSpec((_C, _SPC, _H, _W), lambda i: (0, i, 0, 0)),
        pl.BlockSpec((_C, _FPC, _H, _W), lambda i: (0, i, 0, 0)),
    ),
    out_shape=(
        jax.ShapeDtypeStruct((_C, _TS, _H, _W), jnp.float32),
        jax.ShapeDtypeStruct((_C, _T, _H, _W), jnp.float32),
    ),
)


def kernel(frames):
    return _pack(frames)


# speedup vs baseline: 54.5042x; 1.0625x over previous
import jax
import jax.numpy as jnp
from jax.experimental import pallas as pl

_C, _T, _H, _W = 3, 64, 224, 224
_TS = 16
_NG = 4
_FPC = _T // _NG
_SPC = _TS // _NG


def _body(x_ref, slow_ref, fast_ref):
    fast_ref[...] = x_ref[...]
    k = pl.program_id(0)
    for j in range(_SPC):
        i = _SPC * k + j
        off = (63 * i) // 15 - _FPC * k
        slow_ref[:, pl.ds(j, 1)] = x_ref[:, pl.ds(off, 1)]


_pack = pl.pallas_call(
    _body,
    grid=(_NG,),
    in_specs=[
        pl.BlockSpec((_C, _FPC, _H, _W), lambda i: (0, i, 0, 0)),
    ],
    out_specs=(
        pl.BlockSpec((_C, _SPC, _H, _W), lambda i: (0, i, 0, 0)),
        pl.BlockSpec((_C, _FPC, _H, _W), lambda i: (0, i, 0, 0)),
    ),
    out_shape=(
        jax.ShapeDtypeStruct((_C, _TS, _H, _W), jnp.float32),
        jax.ShapeDtypeStruct((_C, _T, _H, _W), jnp.float32),
    ),
)


def kernel(frames):
    return _pack(frames)
